# baseline (device time: 167763 ns/iter reference)
import jax
import jax.numpy as jnp
from jax import lax
from jax.experimental import pallas as pl
from jax.experimental.pallas import tpu as pltpu

M_SHARD = 8192
HALF = M_SHARD // 2
N = 1024
N_Z = 2
NC = 16
CH = HALF // NC
NTOT = 2 * NC


def kernel(x):
    def body(x_ref, out_ref, f32_buf, bf_buf, in_sems, store_sems,
             z_send_sems, z_recv_sems, x_send_sems, x_recv_sems):
        my_x = lax.axis_index("x")
        my_y = lax.axis_index("y")
        my_z = lax.axis_index("z")
        z_peer = (my_x, my_y, 1 - my_z)
        x_peer = (1 - my_x, my_y, my_z)

        barrier_sem = pltpu.get_barrier_semaphore()
        for peer in (z_peer, x_peer):
            pl.semaphore_signal(
                barrier_sem, inc=1, device_id=peer,
                device_id_type=pl.DeviceIdType.MESH,
            )
        pl.semaphore_wait(barrier_sem, 2)

        def src_row(k):
            if k < NC:
                return my_x * HALF + k * CH
            return (1 - my_x) * HALF + (k - NC) * CH

        def my_half_rows(i):
            return my_z * M_SHARD + my_x * HALF + i * CH

        def z_recv_rows(i):
            return (1 - my_z) * M_SHARD + my_x * HALF + i * CH

        def x_recv_rows(i):
            return (1 - my_z) * M_SHARD + (1 - my_x) * HALF + i * CH

        in_dmas = []

        def start_in(k):
            d = pltpu.make_async_copy(
                x_ref.at[pl.ds(src_row(k), CH), :],
                f32_buf.at[k % 2],
                in_sems.at[k % 2],
            )
            d.start()
            in_dmas.append(d)

        start_in(0)
        store_dmas = []
        z_rdmas = []
        for k in range(NTOT):
            in_dmas[k].wait()
            if k + 1 < NTOT:
                start_in(k + 1)
            bf_buf[k] = f32_buf[k % 2].astype(jnp.bfloat16)
            d = pltpu.make_async_copy(
                bf_buf.at[k],
                out_ref.at[pl.ds(my_z * M_SHARD + src_row(k), CH), :],
                store_sems.at[k],
            )
            d.start()
            store_dmas.append(d)
            if k < NC:
                r = pltpu.make_async_remote_copy(
                    src_ref=bf_buf.at[k],
                    dst_ref=out_ref.at[pl.ds(my_half_rows(k), CH), :],
                    send_sem=z_send_sems.at[k],
                    recv_sem=z_recv_sems.at[k],
                    device_id=z_peer,
                    device_id_type=pl.DeviceIdType.MESH,
                )
                r.start()
                z_rdmas.append(r)

        x_rdmas = []
        for j in range(NC):
            z_rdmas[j].wait_recv()
            r = pltpu.make_async_remote_copy(
                src_ref=out_ref.at[pl.ds(z_recv_rows(j), CH), :],
                dst_ref=out_ref.at[pl.ds(z_recv_rows(j), CH), :],
                send_sem=x_send_sems.at[j],
                recv_sem=x_recv_sems.at[j],
                device_id=x_peer,
                device_id_type=pl.DeviceIdType.MESH,
            )
            r.start()
            x_rdmas.append(r)

        for j in range(NC):
            recv = pltpu.make_async_remote_copy(
                src_ref=out_ref.at[pl.ds(x_recv_rows(j), CH), :],
                dst_ref=out_ref.at[pl.ds(x_recv_rows(j), CH), :],
                send_sem=x_send_sems.at[j],
                recv_sem=x_recv_sems.at[j],
                device_id=x_peer,
                device_id_type=pl.DeviceIdType.MESH,
            )
            recv.wait_recv()
        for j in range(NC):
            z_rdmas[j].wait_send()
            x_rdmas[j].wait_send()
        for d in store_dmas:
            d.wait()

    return pl.pallas_call(
        body,
        out_shape=jax.ShapeDtypeStruct((N_Z * M_SHARD, N), jnp.bfloat16),
        in_specs=[pl.BlockSpec(memory_space=pl.ANY)],
        out_specs=pl.BlockSpec(memory_space=pl.ANY),
        scratch_shapes=[
            pltpu.VMEM((2, CH, N), jnp.float32),
            pltpu.VMEM((NTOT, CH, N), jnp.bfloat16),
            pltpu.SemaphoreType.DMA((2,)),
            pltpu.SemaphoreType.DMA((NTOT,)),
            pltpu.SemaphoreType.DMA((NC,)),
            pltpu.SemaphoreType.DMA((NC,)),
            pltpu.SemaphoreType.DMA((NC,)),
            pltpu.SemaphoreType.DMA((NC,)),
        ],
        compiler_params=pltpu.CompilerParams(collective_id=0),
    )(x)


# device time: 134669 ns/iter; 1.2457x vs baseline; 1.2457x over previous
import jax
import jax.numpy as jnp
from jax import lax
from jax.experimental import pallas as pl
from jax.experimental.pallas import tpu as pltpu

M_SHARD = 8192
HALF = M_SHARD // 2
N = 1024
N_Z = 2
NCV = 8
CCH = HALF // NCV
NC = 16
CH = HALF // NC


def kernel(x):
    def body(x_ref, out_ref, f32_buf, bf_half, bf_other, in_sems,
             store_sems, z_send_sems, z_recv_sems, x_send_sems,
             x_recv_sems):
        my_x = lax.axis_index("x")
        my_y = lax.axis_index("y")
        my_z = lax.axis_index("z")
        z_peer = (my_x, my_y, 1 - my_z)
        x_peer = (1 - my_x, my_y, my_z)

        barrier_sem = pltpu.get_barrier_semaphore()
        for peer in (z_peer, x_peer):
            pl.semaphore_signal(
                barrier_sem, inc=1, device_id=peer,
                device_id_type=pl.DeviceIdType.MESH,
            )
        pl.semaphore_wait(barrier_sem, 2)

        def src_row(k):
            if k < NCV:
                return my_x * HALF + k * CCH
            return (1 - my_x) * HALF + (k - NCV) * CCH

        def my_half_rows(i):
            return my_z * M_SHARD + my_x * HALF + i * CH

        def z_recv_rows(i):
            return (1 - my_z) * M_SHARD + my_x * HALF + i * CH

        def x_recv_rows(i):
            return (1 - my_z) * M_SHARD + (1 - my_x) * HALF + i * CH

        in_dmas = []

        def start_in(k):
            d = pltpu.make_async_copy(
                x_ref.at[pl.ds(src_row(k), CCH), :],
                f32_buf.at[k % 2],
                in_sems.at[k % 2],
            )
            d.start()
            in_dmas.append(d)

        store_dmas = []
        z_rdmas = []

        def convert_step(k):
            in_dmas[k].wait()
            if k + 1 < 2 * NCV:
                start_in(k + 1)
            buf = bf_half if k < NCV else bf_other
            slot = k if k < NCV else k - NCV
            buf[slot] = f32_buf[k % 2].astype(jnp.bfloat16)
            d = pltpu.make_async_copy(
                buf.at[slot],
                out_ref.at[pl.ds(my_z * M_SHARD + src_row(k), CCH), :],
                store_sems.at[k],
            )
            d.start()
            store_dmas.append(d)
            if k < NCV:
                for s in range(2):
                    i = 2 * k + s
                    r = pltpu.make_async_remote_copy(
                        src_ref=bf_half.at[slot, pl.ds(s * CH, CH), :],
                        dst_ref=out_ref.at[pl.ds(my_half_rows(i), CH), :],
                        send_sem=z_send_sems.at[i],
                        recv_sem=z_recv_sems.at[i],
                        device_id=z_peer,
                        device_id_type=pl.DeviceIdType.MESH,
                    )
                    r.start()
                    z_rdmas.append(r)

        start_in(0)
        for k in range(NCV):
            convert_step(k)

        x_rdmas = []
        for j in range(NC):
            z_rdmas[j].wait_recv()
            r = pltpu.make_async_remote_copy(
                src_ref=out_ref.at[pl.ds(z_recv_rows(j), CH), :],
                dst_ref=out_ref.at[pl.ds(z_recv_rows(j), CH), :],
                send_sem=x_send_sems.at[j],
                recv_sem=x_recv_sems.at[j],
                device_id=x_peer,
                device_id_type=pl.DeviceIdType.MESH,
            )
            r.start()
            x_rdmas.append(r)
            if j % 2 == 1:
                convert_step(NCV + j // 2)

        for j in range(NC):
            recv = pltpu.make_async_remote_copy(
                src_ref=out_ref.at[pl.ds(x_recv_rows(j), CH), :],
                dst_ref=out_ref.at[pl.ds(x_recv_rows(j), CH), :],
                send_sem=x_send_sems.at[j],
                recv_sem=x_recv_sems.at[j],
                device_id=x_peer,
                device_id_type=pl.DeviceIdType.MESH,
            )
            recv.wait_recv()
        for j in range(NC):
            z_rdmas[j].wait_send()
            x_rdmas[j].wait_send()
        for d in store_dmas:
            d.wait()

    return pl.pallas_call(
        body,
        out_shape=jax.ShapeDtypeStruct((N_Z * M_SHARD, N), jnp.bfloat16),
        in_specs=[pl.BlockSpec(memory_space=pl.ANY)],
        out_specs=pl.BlockSpec(memory_space=pl.ANY),
        scratch_shapes=[
            pltpu.VMEM((2, CCH, N), jnp.float32),
            pltpu.VMEM((NCV, CCH, N), jnp.bfloat16),
            pltpu.VMEM((NCV, CCH, N), jnp.bfloat16),
            pltpu.SemaphoreType.DMA((2,)),
            pltpu.SemaphoreType.DMA((2 * NCV,)),
            pltpu.SemaphoreType.DMA((NC,)),
            pltpu.SemaphoreType.DMA((NC,)),
            pltpu.SemaphoreType.DMA((NC,)),
            pltpu.SemaphoreType.DMA((NC,)),
        ],
        compiler_params=pltpu.CompilerParams(collective_id=0),
    )(x)
